# baseline (speedup 1.0000x reference)
import jax
import jax.numpy as jnp
from jax import lax
from jax.experimental import pallas as pl
from jax.experimental.pallas import tpu as pltpu
from jax.experimental.pallas import tpu_sc as plsc

_NC = 2
_NS = 16
_NW = _NC * _NS
_CHUNK = 512


def kernel(x, table):
    b, s = x.shape
    n = b * s
    v, d = table.shape
    dw = 2 * d
    idx = x.reshape(n)
    per_w = n // _NW

    tab128 = table.reshape(v // 2, dw)
    jdx = jax.lax.shift_right_logical(idx, 1)

    mesh = plsc.VectorSubcoreMesh(core_axis_name="c", subcore_axis_name="s")

    @pl.kernel(
        out_type=jax.ShapeDtypeStruct((n, dw), table.dtype),
        mesh=mesh,
        scratch_types=[
            pltpu.VMEM((per_w,), jnp.int32),
            pltpu.VMEM((_CHUNK, dw), table.dtype),
            pltpu.SemaphoreType.DMA,
        ],
    )
    def gather_kernel(tab_hbm, idx_hbm, out_hbm, idx_v, rows_v, sem):
        wid = lax.axis_index("s") * _NC + lax.axis_index("c")
        base = wid * per_w
        pltpu.sync_copy(idx_hbm.at[pl.ds(base, per_w)], idx_v)

        @pl.loop(0, per_w, step=_CHUNK)
        def _(off):
            pltpu.async_copy(
                tab_hbm.at[idx_v.at[pl.ds(off, _CHUNK)]], rows_v, sem
            ).wait()
            pltpu.sync_copy(rows_v, out_hbm.at[pl.ds(base + off, _CHUNK)])

    pairs = gather_kernel(tab128, jdx)
    odd = (idx & 1).astype(jnp.bool_)[:, None]
    out = jnp.where(odd, pairs[:, d:], pairs[:, :d])
    return out.reshape(b, s, d)
